# Initial kernel scaffold; baseline (speedup 1.0000x reference)
import jax
import jax.numpy as jnp
from jax.experimental import pallas as pl


def kernel(pha, amp):
    raise NotImplementedError("write your pallas kernel here")


# speedup vs baseline: 83.8160x; 83.8160x over previous
import functools
import math

import jax
import jax.numpy as jnp
import numpy as np
from jax import lax
from jax.experimental import pallas as pl
from jax.experimental.pallas import tpu as pltpu
from jax.experimental.pallas import tpu_sc as plsc

_N_BINS = 18
_EPS = 1e-9
_T = 1024
_LANES = 16
_CHUNKS = _T // _LANES
_N_ROWS = 9
_ROW_WORDS = _N_BINS * _LANES
_TASK_WORDS = _N_ROWS * _ROW_WORDS
_N_TASKS = 128
_N_SBC = 16


def _sc_hist_kernel(pha_hbm, amp_hbm, cut_hbm, out_hbm,
                    pha_v, amp_v, hist_v, cut_v):
    nc = 2
    wid = lax.axis_index("s") * nc + lax.axis_index("c")
    tasks_per_w = _N_TASKS // 32
    base_task = wid * tasks_per_w
    sbc = base_task // 8

    pltpu.sync_copy(cut_hbm, cut_v)
    pltpu.sync_copy(amp_hbm.at[sbc], amp_v)

    lane = lax.iota(jnp.int32, _LANES)
    scale = jnp.float32(_N_BINS / (2.0 * math.pi))
    pi32 = jnp.float32(math.pi)
    ones = jnp.full((_LANES,), 1.0, dtype=jnp.float32)
    zeros = jnp.zeros((_LANES,), dtype=jnp.float32)

    for tt in range(tasks_per_w):
        task = base_task + tt
        pltpu.sync_copy(pha_hbm.at[task], pha_v)

        def _zero_body(i, c):
            hist_v[pl.ds(i * _LANES, _LANES)] = zeros
            return c
        lax.fori_loop(0, _TASK_WORDS // _LANES, _zero_body, 0)

        def _chunk_body(i, c):
            ph = pha_v[pl.ds(i * _LANES, _LANES)]
            raw = (ph + pi32) * scale
            raw = jnp.where(raw > 0.0, raw, 0.0)
            raw = jnp.where(raw < 17.0, raw, 17.0)
            idx0 = raw.astype(jnp.int32)
            c_lo = plsc.load_gather(cut_v, [idx0])
            c_hi = plsc.load_gather(cut_v, [idx0 + 1])
            dec = jnp.logical_and(ph <= c_lo, idx0 > 0).astype(jnp.int32)
            inc = jnp.logical_and(ph > c_hi, idx0 < 17).astype(jnp.int32)
            bidx = idx0 - dec + inc
            base = bidx * _LANES + lane
            for fa in range(8):
                av = amp_v[fa, pl.ds(i * _LANES, _LANES)]
                plsc.addupdate_scatter(hist_v, [base + fa * _ROW_WORDS], av)
            plsc.addupdate_scatter(hist_v, [base + 8 * _ROW_WORDS], ones)
            return c
        lax.fori_loop(0, _CHUNKS, _chunk_body, 0)

        pltpu.sync_copy(hist_v, out_hbm.at[task])


def _sc_hist(pha_t, amp_t, cut_pad):
    mesh = plsc.VectorSubcoreMesh(core_axis_name="c", subcore_axis_name="s")
    f = functools.partial(
        pl.kernel,
        mesh=mesh,
        out_type=jax.ShapeDtypeStruct((_N_TASKS, _TASK_WORDS), jnp.float32),
        scratch_types=[
            pltpu.VMEM((_T,), jnp.float32),
            pltpu.VMEM((8, _T), jnp.float32),
            pltpu.VMEM((_TASK_WORDS,), jnp.float32),
            pltpu.VMEM((24,), jnp.float32),
        ],
        compiler_params=pltpu.CompilerParams(needs_layout_passes=False),
    )(_sc_hist_kernel)
    return f(pha_t, amp_t, cut_pad)


def _tc_finish_kernel(h_ref, o_ref):
    h = h_ref[...]
    sums = jnp.sum(h, axis=-1)
    amp_sums = sums[:, :8, :]
    counts = sums[:, 8:9, :]
    means = amp_sums / (counts + _EPS)
    probs = means / (jnp.sum(means, axis=-1, keepdims=True) + _EPS)
    kl = jnp.sum(probs * jnp.log(probs + _EPS), axis=-1)
    log_n = jnp.float32(np.log(float(_N_BINS)))
    mi = (log_n + kl) / log_n
    mi = 0.5 * (mi[0:64, :] + mi[64:128, :])
    o_ref[...] = jnp.nan_to_num(mi, nan=0.0)


def _tc_finish(h4):
    return pl.pallas_call(
        _tc_finish_kernel,
        out_shape=jax.ShapeDtypeStruct((64, 8), jnp.float32),
    )(h4)


def kernel(pha, amp):
    pha = pha.astype(jnp.float32)
    amp = amp.astype(jnp.float32)
    pha_t = pha.transpose(3, 0, 1, 2, 4).reshape(_N_TASKS, _T)
    amp_t = amp.transpose(3, 0, 1, 2, 4).reshape(_N_SBC, 8, _T)
    cutoffs = jnp.linspace(-np.pi, np.pi, _N_BINS + 1).astype(jnp.float32)
    cut_pad = jnp.concatenate([cutoffs, jnp.zeros((5,), jnp.float32)])
    hist = _sc_hist(pha_t, amp_t, cut_pad)
    h4 = hist.reshape(_N_TASKS, _N_ROWS, _N_BINS, _LANES)
    mi = _tc_finish(h4)
    return mi.reshape(2, 4, 8, 8)


# speedup vs baseline: 120.2844x; 1.4351x over previous
import functools
import math

import jax
import jax.numpy as jnp
import numpy as np
from jax import lax
from jax.experimental import pallas as pl
from jax.experimental.pallas import tpu as pltpu
from jax.experimental.pallas import tpu_sc as plsc

_N_BINS = 18
_EPS = 1e-9
_T = 1024
_LANES = 16
_CHUNKS = _T // _LANES
_N_ROWS = 9
_ROW_WORDS = _N_ROWS * _N_BINS
_TASK_WORDS = _LANES * _ROW_WORDS
_N_TASKS = 128
_N_SBC = 16
_UNROLL = 4


def _sc_hist_kernel(pha_hbm, amp_hbm, cut_hbm, out_hbm,
                    pha_v, amp_v, hist_v, cut_v):
    nc = 2
    wid = lax.axis_index("s") * nc + lax.axis_index("c")
    tasks_per_w = _N_TASKS // 32
    base_task = wid * tasks_per_w
    sbc = base_task // 8

    pltpu.sync_copy(cut_hbm, cut_v)
    pltpu.sync_copy(amp_hbm.at[sbc], amp_v)

    lane162 = lax.iota(jnp.int32, _LANES) * _ROW_WORDS
    scale = jnp.float32(_N_BINS / (2.0 * math.pi))
    pi32 = jnp.float32(math.pi)
    ones = jnp.full((_LANES,), 1.0, dtype=jnp.float32)
    zeros = jnp.zeros((_LANES,), dtype=jnp.float32)

    def _task_body(tt, c):
        task = base_task + tt
        pltpu.sync_copy(pha_hbm.at[task], pha_v)

        def _zero_body(i, c2):
            for k in range(6):
                hist_v[pl.ds((i * 6 + k) * _LANES, _LANES)] = zeros
            return c2
        lax.fori_loop(0, (_TASK_WORDS // _LANES) // 6, _zero_body, 0)

        def _one_chunk(i):
            ph = pha_v[pl.ds(i * _LANES, _LANES)]
            raw = (ph + pi32) * scale
            raw = jnp.minimum(jnp.maximum(raw, 0.0), 17.0)
            idx0 = raw.astype(jnp.int32)
            c_lo = plsc.load_gather(cut_v, [idx0])
            c_hi = plsc.load_gather(cut_v, [idx0 + 1])
            dec = jnp.logical_and(ph <= c_lo, idx0 > 0).astype(jnp.int32)
            inc = jnp.logical_and(ph > c_hi, idx0 < 17).astype(jnp.int32)
            base = lane162 + (idx0 - dec + inc)
            for fa in range(8):
                av = amp_v[fa, pl.ds(i * _LANES, _LANES)]
                plsc.addupdate_scatter(hist_v, [base + fa * _N_BINS], av)
            plsc.addupdate_scatter(hist_v, [base + 8 * _N_BINS], ones)

        def _chunk_body(j, c2):
            for k in range(_UNROLL):
                _one_chunk(j * _UNROLL + k)
            return c2
        lax.fori_loop(0, _CHUNKS // _UNROLL, _chunk_body, 0)

        pltpu.sync_copy(hist_v, out_hbm.at[task])
        return c

    lax.fori_loop(0, tasks_per_w, _task_body, 0)


def _sc_hist(pha_t, amp_t, cut_pad):
    mesh = plsc.VectorSubcoreMesh(core_axis_name="c", subcore_axis_name="s")
    f = functools.partial(
        pl.kernel,
        mesh=mesh,
        out_type=jax.ShapeDtypeStruct((_N_TASKS, _TASK_WORDS), jnp.float32),
        scratch_types=[
            pltpu.VMEM((_T,), jnp.float32),
            pltpu.VMEM((8, _T), jnp.float32),
            pltpu.VMEM((_TASK_WORDS,), jnp.float32),
            pltpu.VMEM((24,), jnp.float32),
        ],
        compiler_params=pltpu.CompilerParams(needs_layout_passes=False),
    )(_sc_hist_kernel)
    return f(pha_t, amp_t, cut_pad)


def _tc_finish_kernel(h_ref, o_ref):
    h = h_ref[...]
    acc = h[:, 0:_ROW_WORDS]
    for l in range(1, _LANES):
        acc = acc + h[:, l * _ROW_WORDS:(l + 1) * _ROW_WORDS]
    counts = acc[:, 8 * _N_BINS:9 * _N_BINS]
    log_n = jnp.float32(np.log(float(_N_BINS)))
    cols = []
    for fa in range(8):
        s_fa = acc[:, fa * _N_BINS:(fa + 1) * _N_BINS]
        mean = s_fa / (counts + _EPS)
        tot = jnp.sum(mean, axis=-1, keepdims=True)
        probs = mean / (tot + _EPS)
        kl = jnp.sum(probs * jnp.log(probs + _EPS), axis=-1, keepdims=True)
        cols.append((log_n + kl) / log_n)
    mi = jnp.concatenate(cols, axis=-1)
    mi = 0.5 * (mi[0:64, :] + mi[64:128, :])
    o_ref[...] = jnp.nan_to_num(mi, nan=0.0)


def _tc_finish(h2):
    return pl.pallas_call(
        _tc_finish_kernel,
        out_shape=jax.ShapeDtypeStruct((64, 8), jnp.float32),
    )(h2)


def kernel(pha, amp):
    pha = pha.astype(jnp.float32)
    amp = amp.astype(jnp.float32)
    pha_t = pha.transpose(3, 0, 1, 2, 4).reshape(_N_TASKS, _T)
    amp_t = amp.transpose(3, 0, 1, 2, 4).reshape(_N_SBC, 8, _T)
    cutoffs = jnp.linspace(-np.pi, np.pi, _N_BINS + 1).astype(jnp.float32)
    cut_pad = jnp.concatenate([cutoffs, jnp.zeros((5,), jnp.float32)])
    hist = _sc_hist(pha_t, amp_t, cut_pad)
    mi = _tc_finish(hist)
    return mi.reshape(2, 4, 8, 8)
